# baseline (speedup 1.0000x reference)
import functools

import jax
import jax.numpy as jnp
from jax import lax
from jax.experimental import pallas as pl
from jax.experimental.pallas import tpu as pltpu
from jax.experimental.pallas import tpu_sc as plsc

B = 4
N = 2048
L = 16
NW = 32
K = 1024
QPW = (B * K) // NW
QG = 8


def _sc_body(pred_hbm, gt_hbm, out_hbm, qv, refv, r2h, outv):
    cid = lax.axis_index("c")
    sid = lax.axis_index("s")
    wid = cid * 16 + sid
    batch = wid // 8
    pos = wid % 8

    pltpu.sync_copy(pred_hbm.at[batch], refv)
    pltpu.sync_copy(gt_hbm.at[batch], qv)

    def r2_body(j, _):
        off = pl.multiple_of(j * L, L)
        rx = refv[0, pl.ds(off, L)]
        ry = refv[1, pl.ds(off, L)]
        rz = refv[2, pl.ds(off, L)]
        r2h[pl.ds(off, L)] = 0.5 * (rx * rx + ry * ry + rz * rz)
        return 0

    lax.fori_loop(0, N // L, r2_body, 0, unroll=2)

    big = jnp.float32(3e38)
    mask0 = lax.iota(jnp.int32, L) == 0

    def group_body(g, _):
        qb = pos * QPW + g * QG
        nqx, nqy, nqz, q2 = [], [], [], []
        for u in range(QG):
            qi = jnp.full((L,), qb + u, jnp.int32)
            xs = plsc.load_gather(qv, [jnp.zeros((L,), jnp.int32), qi])
            ys = plsc.load_gather(qv, [jnp.ones((L,), jnp.int32), qi])
            zs = plsc.load_gather(qv, [jnp.full((L,), 2, jnp.int32), qi])
            nqx.append(-xs)
            nqy.append(-ys)
            nqz.append(-zs)
            q2.append(xs * xs + ys * ys + zs * zs)

        def scan_body(j, accs):
            off = pl.multiple_of(j * L, L)
            rx = refv[0, pl.ds(off, L)]
            ry = refv[1, pl.ds(off, L)]
            rz = refv[2, pl.ds(off, L)]
            rh = r2h[pl.ds(off, L)]
            out = []
            for u in range(QG):
                t = nqx[u] * rx + rh
                t = nqy[u] * ry + t
                t = nqz[u] * rz + t
                out.append(jnp.minimum(accs[u], t))
            return tuple(out)

        accs = lax.fori_loop(
            0, N // L, scan_body, tuple([jnp.full((L,), big)] * QG), unroll=2
        )

        for u in range(QG):
            zsq = jnp.maximum(accs[u] + accs[u] + q2[u], 0.0)
            zmin = lax.reduce_min(zsq, (0,))
            plsc.store_scatter(
                outv,
                [jnp.full((L,), g * QG + u, jnp.int32)],
                jnp.full((L,), zmin),
                mask=mask0,
            )
        return 0

    lax.fori_loop(0, QPW // QG, group_body, 0)

    obase = batch * K + pos * QPW
    pltpu.sync_copy(outv, out_hbm.at[pl.ds(obase, QPW)])


_sc_minsq = functools.partial(
    pl.kernel,
    out_type=jax.ShapeDtypeStruct((B * K,), jnp.float32),
    mesh=plsc.VectorSubcoreMesh(core_axis_name="c", subcore_axis_name="s"),
    compiler_params=pltpu.CompilerParams(
        use_tc_tiling_on_sc=False, needs_layout_passes=False
    ),
    scratch_types=[
        pltpu.VMEM((3, N), jnp.float32),
        pltpu.VMEM((3, N), jnp.float32),
        pltpu.VMEM((N,), jnp.float32),
        pltpu.VMEM((QPW,), jnp.float32),
    ],
)(_sc_body)


QT = 1024


def _tc_nn_body(q_ref, r_ref, o_ref):
    b = pl.program_id(0)
    t = pl.program_id(1)
    qt = jnp.transpose(q_ref[0], (1, 0))
    r = r_ref[0]
    rx = r[0:1, :]
    ry = r[1:2, :]
    rz = r[2:3, :]
    acc = rx * rx + ry * ry + rz * rz
    acc = acc + (-2.0 * qt[:, 0:1]) * rx
    acc = acc + (-2.0 * qt[:, 1:2]) * ry
    acc = acc + (-2.0 * qt[:, 2:3]) * rz
    m = jnp.min(acc, axis=1)
    q2 = jnp.sum(qt * qt, axis=1)
    z = jnp.sqrt(jnp.maximum(m + q2, 0.0))
    s = jnp.sum(z) * jnp.float32(1.0 / (B * N))

    @pl.when(jnp.logical_and(b == 0, t == 0))
    def _():
        o_ref[0, 0] = 0.0

    o_ref[0, 0] += s


def _tc_nn_sum(queries, refs, q_start=0):
    nq = queries.shape[2] - q_start
    toff = q_start // QT
    return pl.pallas_call(
        _tc_nn_body,
        grid=(B, nq // QT),
        in_specs=[
            pl.BlockSpec((1, 3, QT), lambda b, t: (b, 0, t + toff)),
            pl.BlockSpec((1, 3, N), lambda b, t: (b, 0, 0)),
        ],
        out_specs=pl.BlockSpec(memory_space=pltpu.SMEM),
        out_shape=jax.ShapeDtypeStruct((1, 1), jnp.float32),
    )(queries, refs)


def _tc_sqrtsum_body(x_ref, o_ref):
    x = x_ref[...]
    z = jnp.sqrt(jnp.maximum(x, 0.0))
    o_ref[0, 0] = jnp.sum(z) * jnp.float32(1.0 / (B * N))


def kernel(predict_pc, gt_pc):
    minsq = _sc_minsq(
        predict_pc, gt_pc
    )

    t2 = _tc_nn_sum(predict_pc, gt_pc)
    t1b = _tc_nn_sum(gt_pc, predict_pc, q_start=K)

    t1a = pl.pallas_call(
        _tc_sqrtsum_body,
        out_shape=jax.ShapeDtypeStruct((1, 1), jnp.float32),
        out_specs=pl.BlockSpec(memory_space=pltpu.SMEM),
    )(minsq.reshape(16, 256))

    return t1a[0, 0] + t1b[0, 0] + t2[0, 0]


# speedup vs baseline: 1.0001x; 1.0001x over previous
import functools

import jax
import jax.numpy as jnp
from jax import lax
from jax.experimental import pallas as pl
from jax.experimental.pallas import tpu as pltpu
from jax.experimental.pallas import tpu_sc as plsc

B = 4
N = 2048
L = 16
NW = 32
K = 1024
QPW = (B * K) // NW
QG = 8


def _sc_body(pred_hbm, gt_hbm, out_hbm, qv, refv, r2h, outv):
    cid = lax.axis_index("c")
    sid = lax.axis_index("s")
    wid = cid * 16 + sid
    batch = wid // 8
    pos = wid % 8

    pltpu.sync_copy(pred_hbm.at[batch], refv)
    pltpu.sync_copy(gt_hbm.at[batch], qv)

    def r2_body(j, _):
        off = pl.multiple_of(j * L, L)
        rx = refv[0, pl.ds(off, L)]
        ry = refv[1, pl.ds(off, L)]
        rz = refv[2, pl.ds(off, L)]
        r2h[pl.ds(off, L)] = 0.5 * (rx * rx + ry * ry + rz * rz)
        return 0

    lax.fori_loop(0, N // L, r2_body, 0, unroll=2)

    big = jnp.float32(3e38)
    mask0 = lax.iota(jnp.int32, L) == 0

    def group_body(g, _):
        qb = pos * QPW + g * QG
        nqx, nqy, nqz, q2 = [], [], [], []
        for u in range(QG):
            qi = jnp.full((L,), qb + u, jnp.int32)
            xs = plsc.load_gather(qv, [jnp.zeros((L,), jnp.int32), qi])
            ys = plsc.load_gather(qv, [jnp.ones((L,), jnp.int32), qi])
            zs = plsc.load_gather(qv, [jnp.full((L,), 2, jnp.int32), qi])
            nqx.append(-xs)
            nqy.append(-ys)
            nqz.append(-zs)
            q2.append(xs * xs + ys * ys + zs * zs)

        @plsc.parallel_loop(
            0, N // L, unroll=2, carry=tuple([jnp.full((L,), big)] * QG)
        )
        def accs(j, accs_in):
            off = pl.multiple_of(j * L, L)
            rx = refv[0, pl.ds(off, L)]
            ry = refv[1, pl.ds(off, L)]
            rz = refv[2, pl.ds(off, L)]
            rh = r2h[pl.ds(off, L)]
            out = []
            for u in range(QG):
                t = nqx[u] * rx + rh
                t = nqy[u] * ry + t
                t = nqz[u] * rz + t
                out.append(jnp.minimum(accs_in[u], t))
            return tuple(out)

        for u in range(QG):
            zsq = jnp.maximum(accs[u] + accs[u] + q2[u], 0.0)
            zmin = lax.reduce_min(zsq, (0,))
            plsc.store_scatter(
                outv,
                [jnp.full((L,), g * QG + u, jnp.int32)],
                jnp.full((L,), zmin),
                mask=mask0,
            )
        return 0

    lax.fori_loop(0, QPW // QG, group_body, 0)

    obase = batch * K + pos * QPW
    pltpu.sync_copy(outv, out_hbm.at[pl.ds(obase, QPW)])


_sc_minsq = functools.partial(
    pl.kernel,
    out_type=jax.ShapeDtypeStruct((B * K,), jnp.float32),
    mesh=plsc.VectorSubcoreMesh(core_axis_name="c", subcore_axis_name="s"),
    compiler_params=pltpu.CompilerParams(
        use_tc_tiling_on_sc=False, needs_layout_passes=False
    ),
    scratch_types=[
        pltpu.VMEM((3, N), jnp.float32),
        pltpu.VMEM((3, N), jnp.float32),
        pltpu.VMEM((N,), jnp.float32),
        pltpu.VMEM((QPW,), jnp.float32),
    ],
)(_sc_body)


QT = 1024


def _tc_nn_body(q_ref, r_ref, o_ref):
    b = pl.program_id(0)
    t = pl.program_id(1)
    qt = jnp.transpose(q_ref[0], (1, 0))
    r = r_ref[0]
    rx = r[0:1, :]
    ry = r[1:2, :]
    rz = r[2:3, :]
    acc = rx * rx + ry * ry + rz * rz
    acc = acc + (-2.0 * qt[:, 0:1]) * rx
    acc = acc + (-2.0 * qt[:, 1:2]) * ry
    acc = acc + (-2.0 * qt[:, 2:3]) * rz
    m = jnp.min(acc, axis=1)
    q2 = jnp.sum(qt * qt, axis=1)
    z = jnp.sqrt(jnp.maximum(m + q2, 0.0))
    s = jnp.sum(z) * jnp.float32(1.0 / (B * N))

    @pl.when(jnp.logical_and(b == 0, t == 0))
    def _():
        o_ref[0, 0] = 0.0

    o_ref[0, 0] += s


def _tc_nn_sum(queries, refs, q_start=0):
    nq = queries.shape[2] - q_start
    toff = q_start // QT
    return pl.pallas_call(
        _tc_nn_body,
        grid=(B, nq // QT),
        in_specs=[
            pl.BlockSpec((1, 3, QT), lambda b, t: (b, 0, t + toff)),
            pl.BlockSpec((1, 3, N), lambda b, t: (b, 0, 0)),
        ],
        out_specs=pl.BlockSpec(memory_space=pltpu.SMEM),
        out_shape=jax.ShapeDtypeStruct((1, 1), jnp.float32),
    )(queries, refs)


def _tc_sqrtsum_body(x_ref, o_ref):
    x = x_ref[...]
    z = jnp.sqrt(jnp.maximum(x, 0.0))
    o_ref[0, 0] = jnp.sum(z) * jnp.float32(1.0 / (B * N))


def kernel(predict_pc, gt_pc):
    minsq = _sc_minsq(
        predict_pc, gt_pc
    )

    t2 = _tc_nn_sum(predict_pc, gt_pc)
    t1b = _tc_nn_sum(gt_pc, predict_pc, q_start=K)

    t1a = pl.pallas_call(
        _tc_sqrtsum_body,
        out_shape=jax.ShapeDtypeStruct((1, 1), jnp.float32),
        out_specs=pl.BlockSpec(memory_space=pltpu.SMEM),
    )(minsq.reshape(16, 256))

    return t1a[0, 0] + t1b[0, 0] + t2[0, 0]
